# baseline (speedup 1.0000x reference)
import functools

import jax
import jax.numpy as jnp
from jax import lax
from jax.experimental import pallas as pl
from jax.experimental.pallas import tpu as pltpu
from jax.experimental.pallas import tpu_sc as plsc

VOCAB = 100000
EMBED = 64
BATCH = 1024
TV = 2048
NBUF = 3
_G = TV // 8
_NSTEP = (VOCAB + TV - 1) // TV
_TAILG = VOCAB // 8 - (_NSTEP - 1) * _G

_NC = 2
_NS = 16
_NW = _NC * _NS
_BPW = BATCH // _NW


def _sc_gather(table, idx):
    mesh = plsc.VectorSubcoreMesh(core_axis_name="c", subcore_axis_name="s")

    @functools.partial(
        pl.kernel,
        mesh=mesh,
        out_type=jax.ShapeDtypeStruct((BATCH, EMBED), jnp.float32),
        scratch_types=[
            pltpu.VMEM((_BPW,), jnp.int32),
            pltpu.VMEM((_BPW, EMBED), jnp.float32),
            pltpu.SemaphoreType.DMA,
        ],
        compiler_params=pltpu.CompilerParams(use_tc_tiling_on_sc=False),
    )
    def k(table_hbm, idx_hbm, out_hbm, idx_v, rows_v, sem):
        wid = lax.axis_index("s") * _NC + lax.axis_index("c")
        base = wid * _BPW
        pltpu.sync_copy(idx_hbm.at[pl.ds(base, _BPW)], idx_v)
        pltpu.async_copy(table_hbm.at[idx_v], rows_v, sem).wait()
        pltpu.sync_copy(rows_v, out_hbm.at[pl.ds(base, _BPW)])

    return k(table, idx)


def _mm_body(emb_ref, w_ref, b_ref, out_hbm, buf, sems):
    i = pl.program_id(0)
    slot = lax.rem(i, NBUF)

    @pl.when(i >= NBUF)
    def _drain_old():
        j = i - NBUF
        pltpu.make_async_copy(
            buf.at[slot, :, pl.ds(0, 8), :],
            out_hbm.at[pl.ds(j * _G, _G)],
            sems.at[slot],
        ).wait()

    val = lax.dot_general(
        w_ref[...], emb_ref[...],
        (((1,), (1,)), ((), ())),
        preferred_element_type=jnp.float32,
    ) + b_ref[...].T
    buf[slot, :, pl.ds(0, 8), :] = val.reshape(_G, 8, BATCH)

    @pl.when(i < _NSTEP - 1)
    def _start_full():
        pltpu.make_async_copy(
            buf.at[slot, :, pl.ds(0, 8), :],
            out_hbm.at[pl.ds(i * _G, _G)],
            sems.at[slot],
        ).start()

    @pl.when(i == _NSTEP - 1)
    def _start_tail_and_drain_all():
        base = (_NSTEP - 1) * _G
        pltpu.make_async_copy(
            buf.at[slot, pl.ds(0, _TAILG), pl.ds(0, 8), :],
            out_hbm.at[pl.ds(base, _TAILG)],
            sems.at[slot],
        ).start()
        for k in range(min(NBUF - 1, _NSTEP - 1), 0, -1):
            j = _NSTEP - 1 - k
            pltpu.make_async_copy(
                buf.at[j % NBUF, :, pl.ds(0, 8), :],
                out_hbm.at[pl.ds(j * _G, _G)],
                sems.at[j % NBUF],
            ).wait()
        pltpu.make_async_copy(
            buf.at[slot, pl.ds(0, _TAILG), pl.ds(0, 8), :],
            out_hbm.at[pl.ds(base, _TAILG)],
            sems.at[slot],
        ).wait()


def _tc_project_t(embeds, lin_w, lin_b2d):
    return pl.pallas_call(
        _mm_body,
        grid=(_NSTEP,),
        in_specs=[
            pl.BlockSpec((BATCH, EMBED), lambda i: (0, 0)),
            pl.BlockSpec((TV, EMBED), lambda i: (i, 0)),
            pl.BlockSpec((1, TV), lambda i: (0, i)),
        ],
        out_specs=pl.BlockSpec(memory_space=pl.ANY),
        out_shape=jax.ShapeDtypeStruct((VOCAB // 8, 8, BATCH), jnp.float32),
        scratch_shapes=[
            pltpu.VMEM((NBUF, _G, 16, BATCH), jnp.float32),
            pltpu.SemaphoreType.DMA((NBUF,)),
        ],
        compiler_params=pltpu.CompilerParams(vmem_limit_bytes=128 * 1024 * 1024),
    )(embeds, lin_w, lin_b2d)


def kernel(input_word, emb_table, lin_w, lin_b):
    embeds = jnp.take(emb_table, input_word, axis=0)
    out3 = _tc_project_t(embeds, lin_w, lin_b.reshape(1, VOCAB))
    return out3.reshape(VOCAB, BATCH).T
